# Initial kernel scaffold; baseline (speedup 1.0000x reference)
import jax
import jax.numpy as jnp
from jax.experimental import pallas as pl


def kernel(x, edge_index, batch, W0, b0, W1, b1, W2, b2):
    raise NotImplementedError("write your pallas kernel here")


# speedup vs baseline: 2.2374x; 2.2374x over previous
import jax
import jax.numpy as jnp
from jax.experimental import pallas as pl
from jax.experimental.pallas import tpu as pltpu

_G = 128
_BN = 512
_CORES = 2


def _body(xb, bb, w0, b0, w1, b1, w2, b2, sums_ref, cnt_ref):
    j = pl.program_id(1)
    h = jnp.maximum(
        jnp.dot(xb[...], w0[...], preferred_element_type=jnp.float32) + b0[...], 0.0)
    h = jnp.maximum(
        jnp.dot(h, w1[...], preferred_element_type=jnp.float32) + b1[...], 0.0)
    h = jnp.maximum(
        jnp.dot(h, w2[...], preferred_element_type=jnp.float32) + b2[...], 0.0)
    ids = bb[0, 0, :]
    gids = jax.lax.broadcasted_iota(jnp.int32, (_G, _BN), 0)
    onehot = (gids == ids[None, :]).astype(jnp.float32)
    contrib = jnp.dot(onehot, h, preferred_element_type=jnp.float32)
    cnt = jnp.sum(onehot, axis=1, keepdims=True)
    cnt = jnp.broadcast_to(cnt, (_G, 128))

    @pl.when(j == 0)
    def _init():
        sums_ref[0] = contrib
        cnt_ref[0] = cnt

    @pl.when(j > 0)
    def _acc():
        sums_ref[0] += contrib
        cnt_ref[0] += cnt


def kernel(x, edge_index, batch, W0, b0, W1, b1, W2, b2):
    n, d_in = x.shape
    d_h = W0.shape[1]
    d_out = W2.shape[1]

    nb = pl.cdiv(n, _BN * _CORES) * _CORES
    npad = nb * _BN
    xp = jnp.pad(x, ((0, npad - n), (0, 0)))
    ids = jnp.pad(batch.astype(jnp.int32), (0, npad - n), constant_values=_G)
    ids = ids.reshape(nb, 1, _BN)
    nbc = nb // _CORES

    sums, cnts = pl.pallas_call(
        _body,
        grid=(_CORES, nbc),
        in_specs=[
            pl.BlockSpec((_BN, d_in), lambda c, j: (c * nbc + j, 0)),
            pl.BlockSpec((1, 1, _BN), lambda c, j: (c * nbc + j, 0, 0)),
            pl.BlockSpec((d_in, d_h), lambda c, j: (0, 0)),
            pl.BlockSpec((1, d_h), lambda c, j: (0, 0)),
            pl.BlockSpec((d_h, d_h), lambda c, j: (0, 0)),
            pl.BlockSpec((1, d_h), lambda c, j: (0, 0)),
            pl.BlockSpec((d_h, d_out), lambda c, j: (0, 0)),
            pl.BlockSpec((1, d_out), lambda c, j: (0, 0)),
        ],
        out_specs=[
            pl.BlockSpec((1, _G, d_out), lambda c, j: (c, 0, 0)),
            pl.BlockSpec((1, _G, 128), lambda c, j: (c, 0, 0)),
        ],
        out_shape=[
            jax.ShapeDtypeStruct((_CORES, _G, d_out), jnp.float32),
            jax.ShapeDtypeStruct((_CORES, _G, 128), jnp.float32),
        ],
        compiler_params=pltpu.CompilerParams(
            dimension_semantics=("parallel", "arbitrary"),
            vmem_limit_bytes=56 * 1024 * 1024,
        ),
        name="gnnstack_fused",
    )(xp, ids, W0, b0.reshape(1, d_h), W1, b1.reshape(1, d_h),
      W2, b2.reshape(1, d_out))

    total = sums.sum(axis=0)
    count = cnts[:, :, 0].sum(axis=0)
    return total / count[:, None]
